# Initial kernel scaffold; baseline (speedup 1.0000x reference)
import jax
import jax.numpy as jnp
from jax.experimental import pallas as pl


def kernel(input_ids, tld_ids, char_emb, pos_emb, type_emb, tld_emb, W_tld, b_tld, gamma, beta):
    raise NotImplementedError("write your pallas kernel here")


# speedup vs baseline: 8.0126x; 8.0126x over previous
import jax
import jax.numpy as jnp
from jax.experimental import pallas as pl

B, S, H = 4096, 200, 128
V, P, T, TLD, TD = 128, 512, 2, 1000, 64
EPS = 1e-12

TLD_PAD = 1024
BB = 8


def _tld_table_body(tld_emb_ref, w_ref, b_ref, out_ref):
    out_ref[...] = (
        jnp.dot(tld_emb_ref[...], w_ref[...], preferred_element_type=jnp.float32)
        + b_ref[...]
    )


def _main_body(ids_ref, tldid_ref, char_ref, type_ref, pos_ref, tldtab_ref,
               gamma_ref, beta_ref, out_ref):
    ntok = BB * S
    ids = ids_ref[...]
    onehot = (ids == jax.lax.broadcasted_iota(jnp.int32, (1, V), 1)
              ).astype(jnp.float32)
    char_base = char_ref[...] + type_ref[0:1, :]
    emb = jnp.dot(onehot, char_base, preferred_element_type=jnp.float32)
    emb = emb.reshape(BB, S, H) + pos_ref[:S][None, :, :]

    tid = tldid_ref[...]
    tld_onehot = (tid == jax.lax.broadcasted_iota(jnp.int32, (1, TLD_PAD), 1)
                  ).astype(jnp.float32)
    tld_rows = jnp.dot(tld_onehot, tldtab_ref[...],
                       preferred_element_type=jnp.float32)
    emb = emb + tld_rows[:, None, :]

    mean = jnp.mean(emb, axis=-1, keepdims=True)
    var = jnp.mean(jnp.square(emb - mean), axis=-1, keepdims=True)
    out = (emb - mean) * jax.lax.rsqrt(var + EPS)
    out = out * gamma_ref[0][None, None, :] + beta_ref[0][None, None, :]
    out_ref[...] = out


def kernel(input_ids, tld_ids, char_emb, pos_emb, type_emb, tld_emb, W_tld, b_tld, gamma, beta):
    tld_emb_p = jnp.pad(tld_emb, ((0, TLD_PAD - TLD), (0, 0)))
    tld_table = pl.pallas_call(
        _tld_table_body,
        out_shape=jax.ShapeDtypeStruct((TLD_PAD, H), jnp.float32),
    )(tld_emb_p, W_tld, b_tld.reshape(1, H))

    nblk = B // BB
    ids_col = input_ids.astype(jnp.int32).reshape(B * S, 1)
    tld_col = tld_ids.astype(jnp.int32).reshape(B, 1)

    grid_spec = pl.GridSpec(
        grid=(nblk,),
        in_specs=[
            pl.BlockSpec((BB * S, 1), lambda i: (i, 0)),
            pl.BlockSpec((BB, 1), lambda i: (i, 0)),
            pl.BlockSpec((V, H), lambda i: (0, 0)),
            pl.BlockSpec((T, H), lambda i: (0, 0)),
            pl.BlockSpec((P, H), lambda i: (0, 0)),
            pl.BlockSpec((TLD_PAD, H), lambda i: (0, 0)),
            pl.BlockSpec((1, H), lambda i: (0, 0)),
            pl.BlockSpec((1, H), lambda i: (0, 0)),
        ],
        out_specs=pl.BlockSpec((BB, S, H), lambda i: (i, 0, 0)),
    )

    return pl.pallas_call(
        _main_body,
        grid_spec=grid_spec,
        out_shape=jax.ShapeDtypeStruct((B, S, H), jnp.float32),
    )(ids_col, tld_col, char_emb, type_emb, pos_emb,
      tld_table, gamma.reshape(1, H), beta.reshape(1, H))


# speedup vs baseline: 10.9300x; 1.3641x over previous
import jax
import jax.numpy as jnp
from jax.experimental import pallas as pl

B, S, H = 4096, 200, 128
V, P, T, TLD, TD = 128, 512, 2, 1000, 64
EPS = 1e-12

TLD_PAD = 1024
BB = 16


def _tables_body(tld_emb_ref, w_ref, b_ref, char_ref, type_ref, pos_ref,
                 tldtab_ref, charc_ref, posc_ref):
    tld = (jnp.dot(tld_emb_ref[...], w_ref[...],
                   preferred_element_type=jnp.float32) + b_ref[...])
    tldtab_ref[...] = tld - jnp.mean(tld, axis=-1, keepdims=True)
    cb = char_ref[...] + type_ref[0:1, :]
    charc_ref[...] = cb - jnp.mean(cb, axis=-1, keepdims=True)
    p = pos_ref[...]
    posc_ref[...] = p - jnp.mean(p, axis=-1, keepdims=True)


def _main_body(ids_ref, tldid_ref, charc_ref, posc_ref, tldtab_ref,
               gamma_ref, beta_ref, out_ref):
    ids = ids_ref[...]
    onehot = (ids == jax.lax.broadcasted_iota(jnp.int32, (1, V), 1)
              ).astype(jnp.float32)
    emb = jnp.dot(onehot, charc_ref[...], preferred_element_type=jnp.float32)
    emb = emb.reshape(BB, S, H) + posc_ref[:S][None, :, :]

    tid = tldid_ref[...]
    tld_onehot = (tid == jax.lax.broadcasted_iota(jnp.int32, (1, TLD_PAD), 1)
                  ).astype(jnp.float32)
    tld_rows = jnp.dot(tld_onehot, tldtab_ref[...],
                       preferred_element_type=jnp.float32)
    emb = emb + tld_rows[:, None, :]

    var = jnp.mean(jnp.square(emb), axis=-1, keepdims=True)
    scale = jax.lax.rsqrt(var + EPS) * gamma_ref[0][None, None, :]
    out_ref[...] = emb * scale + beta_ref[0][None, None, :]


def kernel(input_ids, tld_ids, char_emb, pos_emb, type_emb, tld_emb, W_tld, b_tld, gamma, beta):
    tld_emb_p = jnp.pad(tld_emb, ((0, TLD_PAD - TLD), (0, 0)))
    tld_table, char_c, pos_c = pl.pallas_call(
        _tables_body,
        out_shape=(
            jax.ShapeDtypeStruct((TLD_PAD, H), jnp.float32),
            jax.ShapeDtypeStruct((V, H), jnp.float32),
            jax.ShapeDtypeStruct((P, H), jnp.float32),
        ),
    )(tld_emb_p, W_tld, b_tld.reshape(1, H), char_emb, type_emb, pos_emb)

    nblk = B // BB
    ids_col = input_ids.astype(jnp.int32).reshape(B * S, 1)
    tld_col = tld_ids.astype(jnp.int32).reshape(B, 1)

    grid_spec = pl.GridSpec(
        grid=(nblk,),
        in_specs=[
            pl.BlockSpec((BB * S, 1), lambda i: (i, 0)),
            pl.BlockSpec((BB, 1), lambda i: (i, 0)),
            pl.BlockSpec((V, H), lambda i: (0, 0)),
            pl.BlockSpec((P, H), lambda i: (0, 0)),
            pl.BlockSpec((TLD_PAD, H), lambda i: (0, 0)),
            pl.BlockSpec((1, H), lambda i: (0, 0)),
            pl.BlockSpec((1, H), lambda i: (0, 0)),
        ],
        out_specs=pl.BlockSpec((BB, S, H), lambda i: (i, 0, 0)),
    )

    return pl.pallas_call(
        _main_body,
        grid_spec=grid_spec,
        out_shape=jax.ShapeDtypeStruct((B, S, H), jnp.float32),
    )(ids_col, tld_col, char_c, pos_c,
      tld_table, gamma.reshape(1, H), beta.reshape(1, H))


# speedup vs baseline: 12.7476x; 1.1663x over previous
import jax
import jax.numpy as jnp
from jax.experimental import pallas as pl

B, S, H = 4096, 200, 128
V, P, T, TLD, TD = 128, 512, 2, 1000, 64
EPS = 1e-12

TLD_PAD = 1024
BB = 32


def _tables_body(tld_emb_ref, w_ref, b_ref, char_ref, type_ref, pos_ref,
                 tldtab_ref, charc_ref, posc_ref):
    tld = (jnp.dot(tld_emb_ref[...], w_ref[...],
                   preferred_element_type=jnp.float32) + b_ref[...])
    tldtab_ref[...] = tld - jnp.mean(tld, axis=-1, keepdims=True)
    cb = char_ref[...] + type_ref[0:1, :]
    charc_ref[...] = cb - jnp.mean(cb, axis=-1, keepdims=True)
    p = pos_ref[...]
    posc_ref[...] = p - jnp.mean(p, axis=-1, keepdims=True)


def _main_body(ids_ref, tldid_ref, charc_ref, posc_ref, tldtab_ref,
               gamma_ref, beta_ref, out_ref):
    ids = ids_ref[...]
    onehot = (ids == jax.lax.broadcasted_iota(jnp.int32, (1, V), 1)
              ).astype(jnp.float32)
    emb = jnp.dot(onehot, charc_ref[...], preferred_element_type=jnp.float32)
    emb = emb.reshape(BB, S, H) + posc_ref[:S][None, :, :]

    tid = tldid_ref[...]
    tld_onehot = (tid == jax.lax.broadcasted_iota(jnp.int32, (1, TLD_PAD), 1)
                  ).astype(jnp.float32)
    tld_rows = jnp.dot(tld_onehot, tldtab_ref[...],
                       preferred_element_type=jnp.float32)
    emb = emb + tld_rows[:, None, :]

    var = jnp.mean(jnp.square(emb), axis=-1, keepdims=True)
    scale = jax.lax.rsqrt(var + EPS) * gamma_ref[0][None, None, :]
    out_ref[...] = emb * scale + beta_ref[0][None, None, :]


def kernel(input_ids, tld_ids, char_emb, pos_emb, type_emb, tld_emb, W_tld, b_tld, gamma, beta):
    tld_emb_p = jnp.pad(tld_emb, ((0, TLD_PAD - TLD), (0, 0)))
    tld_table, char_c, pos_c = pl.pallas_call(
        _tables_body,
        out_shape=(
            jax.ShapeDtypeStruct((TLD_PAD, H), jnp.float32),
            jax.ShapeDtypeStruct((V, H), jnp.float32),
            jax.ShapeDtypeStruct((P, H), jnp.float32),
        ),
    )(tld_emb_p, W_tld, b_tld.reshape(1, H), char_emb, type_emb, pos_emb)

    nblk = B // BB
    ids_col = input_ids.astype(jnp.int32).reshape(B * S, 1)
    tld_col = tld_ids.astype(jnp.int32).reshape(B, 1)

    grid_spec = pl.GridSpec(
        grid=(nblk,),
        in_specs=[
            pl.BlockSpec((BB * S, 1), lambda i: (i, 0)),
            pl.BlockSpec((BB, 1), lambda i: (i, 0)),
            pl.BlockSpec((V, H), lambda i: (0, 0)),
            pl.BlockSpec((P, H), lambda i: (0, 0)),
            pl.BlockSpec((TLD_PAD, H), lambda i: (0, 0)),
            pl.BlockSpec((1, H), lambda i: (0, 0)),
            pl.BlockSpec((1, H), lambda i: (0, 0)),
        ],
        out_specs=pl.BlockSpec((BB, S, H), lambda i: (i, 0, 0)),
    )

    return pl.pallas_call(
        _main_body,
        grid_spec=grid_spec,
        out_shape=jax.ShapeDtypeStruct((B, S, H), jnp.float32),
    )(ids_col, tld_col, char_c, pos_c,
      tld_table, gamma.reshape(1, H), beta.reshape(1, H))


# speedup vs baseline: 14.0359x; 1.1011x over previous
import jax
import jax.numpy as jnp
from jax.experimental import pallas as pl

B, S, H = 4096, 200, 128
V, P, T, TLD, TD = 128, 512, 2, 1000, 64
EPS = 1e-12

TLD_PAD = 1024
BB = 64


def _tables_body(tld_emb_ref, w_ref, b_ref, char_ref, type_ref, pos_ref,
                 tldtab_ref, charc_ref, posc_ref):
    tld = (jnp.dot(tld_emb_ref[...], w_ref[...],
                   preferred_element_type=jnp.float32) + b_ref[...])
    tldtab_ref[...] = tld - jnp.mean(tld, axis=-1, keepdims=True)
    cb = char_ref[...] + type_ref[0:1, :]
    charc_ref[...] = cb - jnp.mean(cb, axis=-1, keepdims=True)
    p = pos_ref[...]
    posc_ref[...] = p - jnp.mean(p, axis=-1, keepdims=True)


def _main_body(ids_ref, tldid_ref, charc_ref, posc_ref, tldtab_ref,
               gamma_ref, beta_ref, out_ref):
    ids = ids_ref[...]
    onehot = (ids == jax.lax.broadcasted_iota(jnp.int32, (1, V), 1)
              ).astype(jnp.float32)
    emb = jnp.dot(onehot, charc_ref[...], preferred_element_type=jnp.float32)
    emb = emb.reshape(BB, S, H) + posc_ref[:S][None, :, :]

    tid = tldid_ref[...]
    tld_onehot = (tid == jax.lax.broadcasted_iota(jnp.int32, (1, TLD_PAD), 1)
                  ).astype(jnp.float32)
    tld_rows = jnp.dot(tld_onehot, tldtab_ref[...],
                       preferred_element_type=jnp.float32)
    emb = emb + tld_rows[:, None, :]

    var = jnp.mean(jnp.square(emb), axis=-1, keepdims=True)
    scale = jax.lax.rsqrt(var + EPS) * gamma_ref[0][None, None, :]
    out_ref[...] = emb * scale + beta_ref[0][None, None, :]


def kernel(input_ids, tld_ids, char_emb, pos_emb, type_emb, tld_emb, W_tld, b_tld, gamma, beta):
    tld_emb_p = jnp.pad(tld_emb, ((0, TLD_PAD - TLD), (0, 0)))
    tld_table, char_c, pos_c = pl.pallas_call(
        _tables_body,
        out_shape=(
            jax.ShapeDtypeStruct((TLD_PAD, H), jnp.float32),
            jax.ShapeDtypeStruct((V, H), jnp.float32),
            jax.ShapeDtypeStruct((P, H), jnp.float32),
        ),
    )(tld_emb_p, W_tld, b_tld.reshape(1, H), char_emb, type_emb, pos_emb)

    nblk = B // BB
    ids_col = input_ids.astype(jnp.int32).reshape(B * S, 1)
    tld_col = tld_ids.astype(jnp.int32).reshape(B, 1)

    grid_spec = pl.GridSpec(
        grid=(nblk,),
        in_specs=[
            pl.BlockSpec((BB * S, 1), lambda i: (i, 0)),
            pl.BlockSpec((BB, 1), lambda i: (i, 0)),
            pl.BlockSpec((V, H), lambda i: (0, 0)),
            pl.BlockSpec((P, H), lambda i: (0, 0)),
            pl.BlockSpec((TLD_PAD, H), lambda i: (0, 0)),
            pl.BlockSpec((1, H), lambda i: (0, 0)),
            pl.BlockSpec((1, H), lambda i: (0, 0)),
        ],
        out_specs=pl.BlockSpec((BB, S, H), lambda i: (i, 0, 0)),
    )

    return pl.pallas_call(
        _main_body,
        grid_spec=grid_spec,
        out_shape=jax.ShapeDtypeStruct((B, S, H), jnp.float32),
    )(ids_col, tld_col, char_c, pos_c,
      tld_table, gamma.reshape(1, H), beta.reshape(1, H))
